# baseline (speedup 1.0000x reference)
import functools

import jax
import jax.numpy as jnp
from jax import lax
from jax.experimental import pallas as pl
from jax.experimental.pallas import tpu as pltpu
from jax.experimental.pallas import tpu_sc as plsc

N = 10000
NIN = 128
FVOL = 27
GROWS = N * FVOL

NC = 2
NT = 16

CH = 13056
NCHUNK = -(-GROWS // CH)
NCHUNK += NCHUNK % 2
CPC = NCHUNK // NC
OUTROWS = NCHUNK * CH
TRASH = CH
STRIPE = CH // NT

EB = 2000
BUF = 128
FLUSH_AT = BUF - 16


def _sc_body(x_hbm, src_hbm, dst_hbm, off_hbm, out_hbm,
             srcv, dstv, offv, srcbuf, gbuf, rowbuf, spmem, sem):
    cid = lax.axis_index("c")
    sid = lax.axis_index("s")
    ept = src_hbm.shape[0] // NT
    nblk = ept // EB
    ebase = sid * ept

    def zero_rowbuf():
        def zrow(r, carry):
            for k in range(NIN // 16):
                rowbuf[r, pl.ds(k * 16, 16)] = jnp.zeros((16,), jnp.float32)
            return carry
        lax.fori_loop(0, BUF, zrow, 0)

    def reset_bufs():
        for k in range(BUF // 16):
            srcbuf[pl.ds(k * 16, 16)] = jnp.zeros((16,), jnp.int32)
            gbuf[pl.ds(k * 16, 16)] = jnp.full((16,), TRASH, jnp.int32)

    def flush():
        pltpu.async_copy(x_hbm.at[srcbuf], rowbuf, sem).wait()
        pltpu.sync_copy(rowbuf, spmem.at[gbuf], add=True)
        reset_bufs()

    reset_bufs()

    def chunk_body(ci, _):
        lo = (cid * CPC + ci) * CH

        zero_rowbuf()
        sbase = sid * STRIPE
        nfull, rem = STRIPE // BUF, STRIPE % BUF
        for j in range(nfull):
            pltpu.sync_copy(rowbuf, spmem.at[pl.ds(sbase + j * BUF, BUF)])
        if rem:
            pltpu.sync_copy(rowbuf.at[pl.ds(0, rem)],
                            spmem.at[pl.ds(sbase + nfull * BUF, rem)])
        plsc.subcore_barrier()

        def blk_body(blk, cnt):
            base = ebase + blk * EB
            pltpu.sync_copy(src_hbm.at[pl.ds(base, EB)], srcv)
            pltpu.sync_copy(dst_hbm.at[pl.ds(base, EB)], dstv)
            pltpu.sync_copy(off_hbm.at[pl.ds(base, EB)], offv)

            def step(i, cnt):
                s16 = srcv[pl.ds(i * 16, 16)]
                d16 = dstv[pl.ds(i * 16, 16)]
                o16 = offv[pl.ds(i * 16, 16)]
                g = d16 * FVOL + o16 - lo
                m = (g >= 0) & (g < CH)
                scan = plsc.cumsum(m.astype(jnp.int32))
                pos = scan + (cnt - 1)
                plsc.store_scatter(srcbuf, [pos], s16, mask=m)
                plsc.store_scatter(gbuf, [pos], g, mask=m)
                cnt2 = cnt + jnp.sum(m.astype(jnp.int32))
                do_flush = cnt2 >= FLUSH_AT

                @pl.when(do_flush)
                def _():
                    flush()

                return jnp.where(do_flush, 0, cnt2)

            return lax.fori_loop(0, EB // 16, step, cnt)

        lax.fori_loop(0, nblk, blk_body, jnp.int32(0))
        flush()
        plsc.subcore_barrier()

        pltpu.sync_copy(spmem.at[pl.ds(sid * STRIPE, STRIPE)],
                        out_hbm.at[pl.ds(lo + sid * STRIPE, STRIPE)])
        plsc.subcore_barrier()
        return _

    lax.fori_loop(0, CPC, chunk_body, 0)


@functools.partial(jax.jit, static_argnames=())
def _run(x, src, dst, off):
    mesh = plsc.VectorSubcoreMesh(core_axis_name="c", subcore_axis_name="s")
    kcall = pl.kernel(
        _sc_body,
        out_type=jax.ShapeDtypeStruct((OUTROWS, NIN), jnp.float32),
        mesh=mesh,
        compiler_params=pltpu.CompilerParams(needs_layout_passes=False),
        scratch_types=[
            pltpu.VMEM((EB,), jnp.int32),
            pltpu.VMEM((EB,), jnp.int32),
            pltpu.VMEM((EB,), jnp.int32),
            pltpu.VMEM((BUF,), jnp.int32),
            pltpu.VMEM((BUF,), jnp.int32),
            pltpu.VMEM((BUF, NIN), jnp.float32),
            pltpu.VMEM_SHARED((CH + 1, NIN), jnp.float32),
            pltpu.SemaphoreType.DMA,
        ],
    )
    return kcall(x, src, dst, off)


def kernel(x, edge_index, edge_offset, weight):
    del weight
    src = edge_index[0].astype(jnp.int32)
    dst = edge_index[1].astype(jnp.int32)
    off = edge_offset.astype(jnp.int32)

    e = src.shape[0]
    epad = -(-e // (NT * EB)) * (NT * EB)
    if epad != e:
        pad = epad - e
        src = jnp.pad(src, (0, pad))
        dst = jnp.pad(dst, (0, pad))
        off = jnp.pad(off, (0, pad), constant_values=-1)

    out = _run(x, src, dst, off)
    return out[:GROWS].reshape(N, FVOL * NIN)
